# baseline (speedup 1.0000x reference)
import functools

import jax
import jax.numpy as jnp
from jax.experimental import pallas as pl
from jax.experimental.pallas import tpu as pltpu

DIM = 96
NUM_HEADS = 3
HEAD_DIM = DIM // NUM_HEADS
KS = 7
R = KS // 2
SCALE = HEAD_DIM ** (-0.5)
HB = 24
GUARD = 128


def _kern(xt_ref, xma_ref, xmb_ref, xmc_ref, xb_ref, spt_ref, spm_ref,
          spb_ref, w_ref, b_ref, out_ref, qkb, bias, spsc, krot7, *, H, W):
    i = pl.program_id(1)
    n = pl.program_id(2)
    P = (HB + 2 * R) * W
    Pc = HB * W
    bf = jnp.bfloat16

    @pl.when(n == 0)
    def _project():
        x2d = jnp.concatenate([
            xt_ref[...].reshape(R * W, DIM),
            xma_ref[...].reshape(HB * W // 3, DIM),
            xmb_ref[...].reshape(HB * W // 3, DIM),
            xmc_ref[...].reshape(HB * W // 3, DIM),
            xb_ref[...].reshape(R * W, DIM),
        ], axis=0).astype(bf)
        qkb[:, :GUARD] = jnp.zeros((2 * DIM, GUARD), bf)
        qkb[:, GUARD + P:] = jnp.zeros((2 * DIM, GUARD), bf)
        qkb[:, GUARD:GUARD + P] = (jax.lax.dot_general(
            w_ref[...], x2d, (((1,), (1,)), ((), ())),
            preferred_element_type=jnp.float32) + b_ref[...]).astype(bf)

        h0 = i * HB
        sp2d = jnp.concatenate([
            spt_ref[...].reshape(R, W),
            spm_ref[...].reshape(HB, W),
            spb_ref[...].reshape(R, W),
        ], axis=0)
        gr = jax.lax.broadcasted_iota(jnp.int32, (HB + 2 * R, W), 0) + (h0 - R)
        sp2d = jnp.where((gr >= 0) & (gr < H), sp2d, -3)
        spsc[...] = sp2d.reshape(1, P)

        spc = spsc[:, R * W:R * W + Pc]
        wwP = jax.lax.broadcasted_iota(
            jnp.int32, (HB + 2 * R, W), 1).reshape(1, P)
        zero = jnp.zeros((1, Pc), jnp.float32)
        neg = jnp.full((1, Pc), -jnp.inf, jnp.float32)
        for dj in range(KS):
            s = dj - R
            if s < 0:
                spro = jnp.pad(spsc[:, :P + s], ((0, 0), (-s, 0)),
                               constant_values=-2)
                spro = jnp.where(wwP >= -s, spro, -2)
            elif s > 0:
                spro = jnp.pad(spsc[:, s:], ((0, 0), (0, s)),
                               constant_values=-2)
                spro = jnp.where(wwP < W - s, spro, -2)
            else:
                spro = spsc[...]
            for di in range(KS):
                sps = spro[:, di * W:di * W + Pc]
                bias[di * 8 + dj:di * 8 + dj + 1, :] = jnp.where(
                    sps == spc, zero, neg).astype(bf)

    q = qkb[pl.ds(n * HEAD_DIM, HEAD_DIM), pl.ds(GUARD + R * W, Pc)]
    S7 = (jax.lax.broadcasted_iota(jnp.int32, (KS, KS * HEAD_DIM), 1)
          // HEAD_DIM
          == jax.lax.broadcasted_iota(
              jnp.int32, (KS, KS * HEAD_DIM), 0)).astype(bf)
    for dj in range(KS):
        if dj != R:
            j = dj if dj < R else dj - 1
            krot7[j] = qkb[pl.ds(DIM + n * HEAD_DIM, HEAD_DIM),
                           pl.ds(GUARD + dj - R, P)]

    def kslice(dj, di):
        if dj == R:
            return qkb[pl.ds(DIM + n * HEAD_DIM, HEAD_DIM),
                       pl.ds(GUARD + di * W, Pc)]
        j = dj if dj < R else dj - 1
        return krot7[j, :, di * W:di * W + Pc]

    blocks = []
    for di in range(KS):
        prods = jnp.concatenate(
            [q * kslice(dj, di) for dj in range(KS)],
            axis=0)
        L7 = jax.lax.dot_general(
            S7, prods, (((1,), (0,)), ((), ())),
            preferred_element_type=jnp.float32)
        blocks.append(L7.astype(bf) + bias[di * 8:di * 8 + KS, :])
    L = jnp.concatenate(blocks, axis=0)
    out_ref[0, 0] = jnp.transpose(L).astype(jnp.float32).reshape(
        HB, W, KS * KS)


def kernel(x, imgSp, qk_w, qk_b):
    B, H, W, C = x.shape
    w_s = jnp.concatenate([qk_w[:DIM] * SCALE, qk_w[DIM:]],
                          axis=0).astype(jnp.bfloat16)
    b_s = jnp.concatenate([qk_b[:DIM] * SCALE, qk_b[DIM:]]).reshape(2 * DIM, 1)
    sp4 = imgSp.reshape(B, H, W // 128, 128)
    nh = H // HB
    hblk = HB // 3
    nrow3 = H // 3 - 1
    P = (HB + 2 * R) * W

    def top_map(b, i, n):
        return (b, jnp.maximum(i * hblk - 1, 0), 0, 0)

    def mid_map(b, i, n):
        return (b, i, 0, 0)

    mrow = HB // 8
    nm8 = H // 8 - 1

    def piece_map(p):
        def _m(b, i, n):
            j = mrow * i + p + jnp.where(n > p, mrow, 0)
            return (b, jnp.minimum(j, nm8), 0, 0)
        return _m

    def bot_map(b, i, n):
        return (b, jnp.minimum((i + 1) * hblk, nrow3), 0, 0)

    out = pl.pallas_call(
        functools.partial(_kern, H=H, W=W),
        grid=(B, nh, NUM_HEADS),
        in_specs=[
            pl.BlockSpec((1, R, W, C), top_map),
            pl.BlockSpec((1, 8, W, C), piece_map(0)),
            pl.BlockSpec((1, 8, W, C), piece_map(1)),
            pl.BlockSpec((1, 8, W, C), piece_map(2)),
            pl.BlockSpec((1, R, W, C), bot_map),
            pl.BlockSpec((1, R, W // 128, 128), top_map),
            pl.BlockSpec((1, HB, W // 128, 128), mid_map),
            pl.BlockSpec((1, R, W // 128, 128), bot_map),
            pl.BlockSpec((2 * DIM, DIM), lambda b, i, n: (0, 0)),
            pl.BlockSpec((2 * DIM, 1), lambda b, i, n: (0, 0)),
        ],
        out_specs=pl.BlockSpec((1, 1, HB, W, KS * KS),
                               lambda b, i, n: (b, n, i, 0, 0)),
        out_shape=jax.ShapeDtypeStruct((B, NUM_HEADS, H, W, KS * KS),
                                       jnp.float32),
        scratch_shapes=[
            pltpu.VMEM((2 * DIM, P + 2 * GUARD), jnp.bfloat16),
            pltpu.VMEM((KS * 8, HB * W), jnp.bfloat16),
            pltpu.VMEM((1, P), jnp.int32),
            pltpu.VMEM((KS - 1, HEAD_DIM, P), jnp.bfloat16),
        ],
    )(x, x, x, x, x, sp4, sp4, sp4, w_s, b_s)
    return out


# speedup vs baseline: 1.0051x; 1.0051x over previous
import functools

import jax
import jax.numpy as jnp
from jax.experimental import pallas as pl
from jax.experimental.pallas import tpu as pltpu

DIM = 96
NUM_HEADS = 3
HEAD_DIM = DIM // NUM_HEADS
KS = 7
R = KS // 2
SCALE = HEAD_DIM ** (-0.5)
HB = 24
GUARD = 128


def _kern(xt_ref, xm_ref, xb_ref, spt_ref, spm_ref, spb_ref, w_ref, b_ref,
          out_ref, qkb, bias, spsc, krot7, *, H, W):
    i = pl.program_id(1)
    n = pl.program_id(2)
    P = (HB + 2 * R) * W
    Pc = HB * W
    bf = jnp.bfloat16

    @pl.when(n == 0)
    def _project():
        x2d = jnp.concatenate([
            xt_ref[...].reshape(R * W, DIM),
            xm_ref[...].reshape(HB * W, DIM),
            xb_ref[...].reshape(R * W, DIM),
        ], axis=0).astype(bf)
        qkb[:, :GUARD] = jnp.zeros((2 * DIM, GUARD), bf)
        qkb[:, GUARD + P:] = jnp.zeros((2 * DIM, GUARD), bf)
        qkb[:, GUARD:GUARD + P] = (jax.lax.dot_general(
            w_ref[...], x2d, (((1,), (1,)), ((), ())),
            preferred_element_type=jnp.float32) + b_ref[...]).astype(bf)

        h0 = i * HB
        sp2d = jnp.concatenate([
            spt_ref[...].reshape(R, W),
            spm_ref[...].reshape(HB, W),
            spb_ref[...].reshape(R, W),
        ], axis=0)
        gr = jax.lax.broadcasted_iota(jnp.int32, (HB + 2 * R, W), 0) + (h0 - R)
        sp2d = jnp.where((gr >= 0) & (gr < H), sp2d, -3)
        spsc[...] = sp2d.reshape(1, P)

        spc = spsc[:, R * W:R * W + Pc]
        wwP = jax.lax.broadcasted_iota(
            jnp.int32, (HB + 2 * R, W), 1).reshape(1, P)
        zero = jnp.zeros((1, Pc), jnp.float32)
        neg = jnp.full((1, Pc), -jnp.inf, jnp.float32)
        for dj in range(KS):
            s = dj - R
            if s < 0:
                spro = jnp.pad(spsc[:, :P + s], ((0, 0), (-s, 0)),
                               constant_values=-2)
                spro = jnp.where(wwP >= -s, spro, -2)
            elif s > 0:
                spro = jnp.pad(spsc[:, s:], ((0, 0), (0, s)),
                               constant_values=-2)
                spro = jnp.where(wwP < W - s, spro, -2)
            else:
                spro = spsc[...]
            for di in range(KS):
                sps = spro[:, di * W:di * W + Pc]
                bias[di * 8 + dj:di * 8 + dj + 1, :] = jnp.where(
                    sps == spc, zero, neg).astype(bf)

    q = qkb[pl.ds(n * HEAD_DIM, HEAD_DIM), pl.ds(GUARD + R * W, Pc)]
    S7 = (jax.lax.broadcasted_iota(jnp.int32, (KS, KS * HEAD_DIM), 1)
          // HEAD_DIM
          == jax.lax.broadcasted_iota(
              jnp.int32, (KS, KS * HEAD_DIM), 0)).astype(bf)
    for dj in range(KS):
        if dj != R:
            j = dj if dj < R else dj - 1
            krot7[j] = qkb[pl.ds(DIM + n * HEAD_DIM, HEAD_DIM),
                           pl.ds(GUARD + dj - R, P)]

    def kslice(dj, di):
        if dj == R:
            return qkb[pl.ds(DIM + n * HEAD_DIM, HEAD_DIM),
                       pl.ds(GUARD + di * W, Pc)]
        j = dj if dj < R else dj - 1
        return krot7[j, :, di * W:di * W + Pc]

    blocks = []
    for di in range(KS):
        prods = jnp.concatenate(
            [q * kslice(dj, di) for dj in range(KS)],
            axis=0)
        L7 = jax.lax.dot_general(
            S7, prods, (((1,), (0,)), ((), ())),
            preferred_element_type=jnp.float32)
        blocks.append(L7.astype(bf) + bias[di * 8:di * 8 + KS, :])
    L = jnp.concatenate(blocks, axis=0)
    out_ref[0, 0] = jnp.transpose(L).astype(jnp.float32).reshape(
        HB, W, KS * KS)


def kernel(x, imgSp, qk_w, qk_b):
    B, H, W, C = x.shape
    w_s = jnp.concatenate([qk_w[:DIM] * SCALE, qk_w[DIM:]],
                          axis=0).astype(jnp.bfloat16)
    b_s = jnp.concatenate([qk_b[:DIM] * SCALE, qk_b[DIM:]]).reshape(2 * DIM, 1)
    sp4 = imgSp.reshape(B, H, W // 128, 128)
    nh = H // HB
    hblk = HB // 3
    nrow3 = H // 3 - 1
    P = (HB + 2 * R) * W

    def top_map(b, i, n):
        return (b, jnp.maximum(i * hblk - 1, 0), 0, 0)

    def mid_map(b, i, n):
        return (b, i, 0, 0)

    def bot_map(b, i, n):
        return (b, jnp.minimum((i + 1) * hblk, nrow3), 0, 0)

    out = pl.pallas_call(
        functools.partial(_kern, H=H, W=W),
        grid=(B, nh, NUM_HEADS),
        in_specs=[
            pl.BlockSpec((1, R, W, C), top_map),
            pl.BlockSpec((1, HB, W, C), mid_map),
            pl.BlockSpec((1, R, W, C), bot_map),
            pl.BlockSpec((1, R, W // 128, 128), top_map),
            pl.BlockSpec((1, HB, W // 128, 128), mid_map),
            pl.BlockSpec((1, R, W // 128, 128), bot_map),
            pl.BlockSpec((2 * DIM, DIM), lambda b, i, n: (0, 0)),
            pl.BlockSpec((2 * DIM, 1), lambda b, i, n: (0, 0)),
        ],
        out_specs=pl.BlockSpec((1, 1, HB, W, KS * KS),
                               lambda b, i, n: (b, n, i, 0, 0)),
        out_shape=jax.ShapeDtypeStruct((B, NUM_HEADS, H, W, KS * KS),
                                       jnp.float32),
        scratch_shapes=[
            pltpu.VMEM((2 * DIM, P + 2 * GUARD), jnp.bfloat16),
            pltpu.VMEM((KS * 8, HB * W), jnp.bfloat16),
            pltpu.VMEM((1, P), jnp.int32),
            pltpu.VMEM((KS - 1, HEAD_DIM, P), jnp.bfloat16),
        ],
    )(x, x, x, sp4, sp4, sp4, w_s, b_s)
    return out


# speedup vs baseline: 1.0121x; 1.0070x over previous
import functools

import jax
import jax.numpy as jnp
from jax.experimental import pallas as pl
from jax.experimental.pallas import tpu as pltpu

DIM = 96
NUM_HEADS = 3
HEAD_DIM = DIM // NUM_HEADS
KS = 7
R = KS // 2
SCALE = HEAD_DIM ** (-0.5)
HB = 24
GUARD = 128


def _kern(xt_ref, xm_ref, xb_ref, spt_ref, spm_ref, spb_ref, w_ref, b_ref,
          out_ref, qkb, bias, spsc, krot7, *, H, W):
    i = pl.program_id(1)
    n = pl.program_id(2)
    P = (HB + 2 * R) * W
    Pc = HB * W
    bf = jnp.bfloat16

    @pl.when(n == 0)
    def _project():
        qkb[:, :GUARD] = jnp.zeros((2 * DIM, GUARD), bf)
        qkb[:, GUARD + P:] = jnp.zeros((2 * DIM, GUARD), bf)

        @pl.when(i == 0)
        def _full():
            x2d = jnp.concatenate([
                xt_ref[...].reshape(R * W, DIM),
                xm_ref[...].reshape(HB * W, DIM),
                xb_ref[...].reshape(R * W, DIM),
            ], axis=0).astype(bf)
            qkb[:, GUARD:GUARD + P] = (jax.lax.dot_general(
                w_ref[...], x2d, (((1,), (1,)), ((), ())),
                preferred_element_type=jnp.float32) + b_ref[...]).astype(bf)

        @pl.when(i > 0)
        def _incremental():
            qkb[:, GUARD:GUARD + 2 * R * W] = qkb[:, GUARD + HB * W:GUARD + P]
            x2d = jnp.concatenate([
                xm_ref[0, R:].reshape((HB - R) * W, DIM),
                xb_ref[...].reshape(R * W, DIM),
            ], axis=0).astype(bf)
            qkb[:, GUARD + 2 * R * W:GUARD + P] = (jax.lax.dot_general(
                w_ref[...], x2d, (((1,), (1,)), ((), ())),
                preferred_element_type=jnp.float32) + b_ref[...]).astype(bf)

        h0 = i * HB
        sp2d = jnp.concatenate([
            spt_ref[...].reshape(R, W),
            spm_ref[...].reshape(HB, W),
            spb_ref[...].reshape(R, W),
        ], axis=0)
        gr = jax.lax.broadcasted_iota(jnp.int32, (HB + 2 * R, W), 0) + (h0 - R)
        sp2d = jnp.where((gr >= 0) & (gr < H), sp2d, -3)
        spsc[...] = sp2d.reshape(1, P)

        spc = spsc[:, R * W:R * W + Pc]
        wwP = jax.lax.broadcasted_iota(
            jnp.int32, (HB + 2 * R, W), 1).reshape(1, P)
        zero = jnp.zeros((1, Pc), jnp.float32)
        neg = jnp.full((1, Pc), -jnp.inf, jnp.float32)
        for dj in range(KS):
            s = dj - R
            if s < 0:
                spro = jnp.pad(spsc[:, :P + s], ((0, 0), (-s, 0)),
                               constant_values=-2)
                spro = jnp.where(wwP >= -s, spro, -2)
            elif s > 0:
                spro = jnp.pad(spsc[:, s:], ((0, 0), (0, s)),
                               constant_values=-2)
                spro = jnp.where(wwP < W - s, spro, -2)
            else:
                spro = spsc[...]
            for di in range(KS):
                sps = spro[:, di * W:di * W + Pc]
                bias[di * 8 + dj:di * 8 + dj + 1, :] = jnp.where(
                    sps == spc, zero, neg).astype(bf)

    q = qkb[pl.ds(n * HEAD_DIM, HEAD_DIM), pl.ds(GUARD + R * W, Pc)]
    S7 = (jax.lax.broadcasted_iota(jnp.int32, (KS, KS * HEAD_DIM), 1)
          // HEAD_DIM
          == jax.lax.broadcasted_iota(
              jnp.int32, (KS, KS * HEAD_DIM), 0)).astype(bf)
    for dj in range(KS):
        if dj != R:
            j = dj if dj < R else dj - 1
            krot7[j] = qkb[pl.ds(DIM + n * HEAD_DIM, HEAD_DIM),
                           pl.ds(GUARD + dj - R, P)]

    def kslice(dj, di):
        if dj == R:
            return qkb[pl.ds(DIM + n * HEAD_DIM, HEAD_DIM),
                       pl.ds(GUARD + di * W, Pc)]
        j = dj if dj < R else dj - 1
        return krot7[j, :, di * W:di * W + Pc]

    blocks = []
    for di in range(KS):
        prods = jnp.concatenate(
            [q * kslice(dj, di) for dj in range(KS)],
            axis=0)
        L7 = jax.lax.dot_general(
            S7, prods, (((1,), (0,)), ((), ())),
            preferred_element_type=jnp.float32)
        blocks.append(L7.astype(bf) + bias[di * 8:di * 8 + KS, :])
    L = jnp.concatenate(blocks, axis=0)
    out_ref[0, 0] = jnp.transpose(L).astype(jnp.float32).reshape(
        HB, W, KS * KS)


def kernel(x, imgSp, qk_w, qk_b):
    B, H, W, C = x.shape
    w_s = jnp.concatenate([qk_w[:DIM] * SCALE, qk_w[DIM:]],
                          axis=0).astype(jnp.bfloat16)
    b_s = jnp.concatenate([qk_b[:DIM] * SCALE, qk_b[DIM:]]).reshape(2 * DIM, 1)
    sp4 = imgSp.reshape(B, H, W // 128, 128)
    nh = H // HB
    hblk = HB // 3
    nrow3 = H // 3 - 1
    P = (HB + 2 * R) * W

    def top_map(b, i, n):
        return (b, 0, 0, 0)

    def sptop_map(b, i, n):
        return (b, jnp.maximum(i * hblk - 1, 0), 0, 0)

    def mid_map(b, i, n):
        return (b, i, 0, 0)

    def bot_map(b, i, n):
        return (b, jnp.minimum((i + 1) * hblk, nrow3), 0, 0)

    out = pl.pallas_call(
        functools.partial(_kern, H=H, W=W),
        grid=(B, nh, NUM_HEADS),
        in_specs=[
            pl.BlockSpec((1, R, W, C), top_map),
            pl.BlockSpec((1, HB, W, C), mid_map),
            pl.BlockSpec((1, R, W, C), bot_map),
            pl.BlockSpec((1, R, W // 128, 128), sptop_map),
            pl.BlockSpec((1, HB, W // 128, 128), mid_map),
            pl.BlockSpec((1, R, W // 128, 128), bot_map),
            pl.BlockSpec((2 * DIM, DIM), lambda b, i, n: (0, 0)),
            pl.BlockSpec((2 * DIM, 1), lambda b, i, n: (0, 0)),
        ],
        out_specs=pl.BlockSpec((1, 1, HB, W, KS * KS),
                               lambda b, i, n: (b, n, i, 0, 0)),
        out_shape=jax.ShapeDtypeStruct((B, NUM_HEADS, H, W, KS * KS),
                                       jnp.float32),
        scratch_shapes=[
            pltpu.VMEM((2 * DIM, P + 2 * GUARD), jnp.bfloat16),
            pltpu.VMEM((KS * 8, HB * W), jnp.bfloat16),
            pltpu.VMEM((1, P), jnp.int32),
            pltpu.VMEM((KS - 1, HEAD_DIM, P), jnp.bfloat16),
        ],
    )(x, x, x, sp4, sp4, sp4, w_s, b_s)
    return out
